# baseline (speedup 1.0000x reference)
import jax
import jax.numpy as jnp
from jax.experimental import pallas as pl
from jax.experimental.pallas import tpu as pltpu

B, N, L, D = 4, 4096, 12, 64
LD = L * D
BD = B * D
BN = B * N

CHX = 256
SX = 2
NXC = BN // CHX
CA = 128
SA = 4
NA = N // CA
CO = 256
NO = N // CO
ROWS_PER_B = N // CHX

PB = NXC
PC = NXC + NA
PD = PC + 1
GRID = PD + NO

_bf16 = jnp.bfloat16
_f32 = jnp.float32


def _mega_kernel(x_any, adj_any, wm2_ref, bm2_ref, wg1_ref, bt1_ref,
                 wg2_ref, bt2_ref, wg3_ref, bt3_ref, wm1_ref, bm1_ref,
                 o_ref, adj_bf, z1, z2, xbuf, astage, xsems, asems):
    i = pl.program_id(0)

    def start_x(ci, slot):
        for s in range(SX):
            sr = CHX // SX
            pltpu.make_async_copy(
                x_any.at[pl.ds(ci * CHX + s * sr, sr), :, :],
                xbuf.at[slot, pl.ds(s * sr, sr), :, :],
                xsems.at[slot, s]).start()

    def wait_x(ci, slot):
        for s in range(SX):
            sr = CHX // SX
            pltpu.make_async_copy(
                x_any.at[pl.ds(ci * CHX + s * sr, sr), :, :],
                xbuf.at[slot, pl.ds(s * sr, sr), :, :],
                xsems.at[slot, s]).wait()

    def start_adj(ci, slot):
        for s in range(SA):
            sr = CA // SA
            pltpu.make_async_copy(
                adj_any.at[pl.ds(ci * CA + s * sr, sr), :],
                astage.at[slot, pl.ds(s * sr, sr), :],
                asems.at[slot, s]).start()

    def wait_adj(ci, slot):
        for s in range(SA):
            sr = CA // SA
            pltpu.make_async_copy(
                adj_any.at[pl.ds(ci * CA + s * sr, sr), :],
                astage.at[slot, pl.ds(s * sr, sr), :],
                asems.at[slot, s]).wait()

    @pl.when(i == 0)
    def _prologue():
        start_x(0, 0)

    def phase_a(slot):
        ci = i

        @pl.when(ci + 1 < NXC)
        def _():
            start_x(ci + 1, 1 - slot)

        wait_x(ci, slot)
        bi_idx = ci // ROWS_PER_B
        n0 = (ci % ROWS_PER_B) * CHX
        rows = pl.ds(n0, CHX)
        wg1 = wg1_ref[:].astype(_bf16)
        bm2 = bm2_ref[:]
        acc = bm2
        for l in range(L):
            xl = xbuf[slot, :, l, :].astype(_bf16)
            wl = wm2_ref[l * D:(l + 1) * D, :].astype(_bf16)
            acc = acc + jnp.dot(xl, wl, preferred_element_type=_f32)
        hcb = jnp.maximum(acc, 0.0).astype(_bf16)
        z1c = jnp.dot(hcb, wg1, preferred_element_type=_f32).astype(_bf16)
        for bi in range(B):
            @pl.when(bi_idx == bi)
            def _():
                z1[rows, bi * D:(bi + 1) * D] = z1c

    @pl.when(i == NXC - 1)
    def _adj_prologue():
        start_adj(0, 0)

    def phase_b(slot):
        ci = i - PB

        @pl.when(ci + 1 < NA)
        def _():
            start_adj(ci + 1, 1 - slot)

        wait_adj(ci, slot)
        rows = pl.ds(ci * CA, CA)
        a_rows = astage[slot].astype(_bf16)
        adj_bf[rows, :] = a_rows
        agg = jnp.dot(a_rows, z1[:], preferred_element_type=_f32)
        h1 = jnp.maximum(agg + bt1_ref[:], 0.0).astype(_bf16)
        wg2 = wg2_ref[:].astype(_bf16)
        for bi in range(B):
            bsl = slice(bi * D, (bi + 1) * D)
            z2[rows, bsl] = jnp.dot(h1[:, bsl], wg2,
                                    preferred_element_type=_f32).astype(_bf16)

    @pl.when(jnp.logical_and(i < NXC, i % 2 == 0))
    def _pa_even():
        phase_a(0)

    @pl.when(jnp.logical_and(i < NXC, i % 2 == 1))
    def _pa_odd():
        phase_a(1)

    @pl.when(jnp.logical_and(jnp.logical_and(i >= PB, i < PC), i % 2 == 0))
    def _pb_even():
        phase_b(0)

    @pl.when(jnp.logical_and(jnp.logical_and(i >= PB, i < PC), i % 2 == 1))
    def _pb_odd():
        phase_b(1)

    @pl.when(i == PC)
    def _phase_c():
        bt2 = bt2_ref[:]
        wg3 = wg3_ref[:].astype(_bf16)
        for rc in range(4):
            rs = slice(rc * (N // 4), (rc + 1) * (N // 4))
            agg = jnp.dot(adj_bf[rs, :], z2[:], preferred_element_type=_f32)
            h2 = jnp.maximum(agg + bt2, 0.0).astype(_bf16)
            for bi in range(B):
                bsl = slice(bi * D, (bi + 1) * D)
                z1[rs, bsl] = jnp.dot(h2[:, bsl], wg3,
                                      preferred_element_type=_f32).astype(_bf16)

    @pl.when(i >= PD)
    def _phase_d():
        ri = i - PD
        rows = pl.ds(ri * CO, CO)
        bt3 = bt3_ref[:]
        bm1 = bm1_ref[:]
        wm1 = wm1_ref[:].astype(_bf16)
        agg = jnp.dot(adj_bf[rows, :], z1[:], preferred_element_type=_f32)
        h3c = jnp.maximum(agg + bt3, 0.0).astype(_bf16)
        for bi in range(B):
            hb = h3c[:, bi * D:(bi + 1) * D]
            o = jnp.dot(hb, wm1, preferred_element_type=_f32) + bm1
            o_ref[bi] = jnp.maximum(o, 0.0)


def kernel(x, adj, W_mlp2, b_mlp2, W_g1, b_g1, W_g2, b_g2, W_g3, b_g3,
           W_mlp1, b_mlp1):
    x3 = x.reshape(BN, L, D)
    bm2 = b_mlp2.reshape(1, D)
    bt = [jnp.tile(b, B).reshape(1, BD) for b in (b_g1, b_g2, b_g3)]
    bm1 = b_mlp1.reshape(1, LD)

    out = pl.pallas_call(
        _mega_kernel,
        grid=(GRID,),
        in_specs=[
            pl.BlockSpec(memory_space=pltpu.MemorySpace.HBM),
            pl.BlockSpec(memory_space=pltpu.MemorySpace.HBM),
            pl.BlockSpec((LD, D), lambda i: (0, 0)),
            pl.BlockSpec((1, D), lambda i: (0, 0)),
            pl.BlockSpec((D, D), lambda i: (0, 0)),
            pl.BlockSpec((1, BD), lambda i: (0, 0)),
            pl.BlockSpec((D, D), lambda i: (0, 0)),
            pl.BlockSpec((1, BD), lambda i: (0, 0)),
            pl.BlockSpec((D, D), lambda i: (0, 0)),
            pl.BlockSpec((1, BD), lambda i: (0, 0)),
            pl.BlockSpec((D, LD), lambda i: (0, 0)),
            pl.BlockSpec((1, LD), lambda i: (0, 0)),
        ],
        out_specs=pl.BlockSpec(
            (B, CO, LD), lambda i: (0, jnp.maximum(i - PD, 0), 0)),
        out_shape=jax.ShapeDtypeStruct((B, N, LD), _f32),
        scratch_shapes=[
            pltpu.VMEM((N, N), _bf16),
            pltpu.VMEM((N, BD), _bf16),
            pltpu.VMEM((N, BD), _bf16),
            pltpu.VMEM((2, CHX, L, D), _f32),
            pltpu.VMEM((2, CA, N), _f32),
            pltpu.SemaphoreType.DMA((2, SX)),
            pltpu.SemaphoreType.DMA((2, SA)),
        ],
    )(x3, adj, W_mlp2, bm2, W_g1, bt[0], W_g2, bt[1], W_g3, bt[2],
      W_mlp1, bm1)
    return out


# speedup vs baseline: 1.7185x; 1.7185x over previous
import jax
import jax.numpy as jnp
from jax.experimental import pallas as pl
from jax.experimental.pallas import tpu as pltpu

B, N, L, D = 4, 4096, 12, 64
LD = L * D
BD = B * D
BM = 512

_bf16 = jnp.bfloat16
_f32 = jnp.float32


def _mlp_in_kernel(x_ref, w_ref, b_ref, o_ref):
    w = w_ref[:].astype(_bf16)
    bias = b_ref[:]
    for bi in range(B):
        xb = x_ref[bi].astype(_bf16)
        h = jnp.dot(xb, w, preferred_element_type=_f32) + bias
        o_ref[:, bi * D:(bi + 1) * D] = jnp.maximum(h, 0.0).astype(_bf16)


def _gcn_kernel(a_ref, h_ref, w1_ref, b1_ref, w2_ref, b2_ref, w3_ref, b3_ref,
                o_ref, z_ref, h2_ref):
    RC = 1024
    layers = ((w1_ref, b1_ref), (w2_ref, b2_ref), (w3_ref, b3_ref))
    src = h_ref
    for li, (w_ref, b_ref) in enumerate(layers):
        w = w_ref[:].astype(_bf16)
        for bi in range(B):
            sl = slice(bi * D, (bi + 1) * D)
            z_ref[:, sl] = jnp.dot(src[:, sl], w,
                                   preferred_element_type=_f32).astype(_bf16)
        dst = o_ref if li == 2 else h2_ref
        for rc in range(N // RC):
            rs = slice(rc * RC, (rc + 1) * RC)
            agg = jnp.dot(a_ref[rs], z_ref[:], preferred_element_type=_f32)
            dst[rs] = jnp.maximum(agg + b_ref[:], 0.0).astype(_bf16)
        src = h2_ref


def _mlp_out_kernel(h_ref, w_ref, b_ref, o_ref):
    w = w_ref[:].astype(_bf16)
    bias = b_ref[:]
    for bi in range(B):
        hb = h_ref[:, bi * D:(bi + 1) * D]
        o = jnp.dot(hb, w, preferred_element_type=_f32) + bias
        o_ref[bi] = jnp.maximum(o, 0.0)


def kernel(x, adj, W_mlp2, b_mlp2, W_g1, b_g1, W_g2, b_g2, W_g3, b_g3,
           W_mlp1, b_mlp1):
    xf = x.reshape(B, N, LD)
    adj_bf = adj.astype(_bf16)
    b2 = b_mlp2.reshape(1, D)
    bt = [jnp.tile(b, B).reshape(1, BD) for b in (b_g1, b_g2, b_g3)]
    b1 = b_mlp1.reshape(1, LD)

    h0 = pl.pallas_call(
        _mlp_in_kernel,
        grid=(N // BM,),
        in_specs=[
            pl.BlockSpec((B, BM, LD), lambda i: (0, i, 0)),
            pl.BlockSpec((LD, D), lambda i: (0, 0)),
            pl.BlockSpec((1, D), lambda i: (0, 0)),
        ],
        out_specs=pl.BlockSpec((BM, BD), lambda i: (i, 0)),
        out_shape=jax.ShapeDtypeStruct((N, BD), _bf16),
    )(xf, W_mlp2, b2)

    h3 = pl.pallas_call(
        _gcn_kernel,
        in_specs=[pl.BlockSpec(memory_space=pltpu.VMEM)] * 8,
        out_specs=pl.BlockSpec(memory_space=pltpu.VMEM),
        out_shape=jax.ShapeDtypeStruct((N, BD), _bf16),
        scratch_shapes=[pltpu.VMEM((N, BD), _bf16),
                        pltpu.VMEM((N, BD), _bf16)],
    )(adj_bf, h0, W_g1, bt[0], W_g2, bt[1], W_g3, bt[2])

    out = pl.pallas_call(
        _mlp_out_kernel,
        grid=(N // BM,),
        in_specs=[
            pl.BlockSpec((BM, BD), lambda i: (i, 0)),
            pl.BlockSpec((D, LD), lambda i: (0, 0)),
            pl.BlockSpec((1, LD), lambda i: (0, 0)),
        ],
        out_specs=pl.BlockSpec((B, BM, LD), lambda i: (0, i, 0)),
        out_shape=jax.ShapeDtypeStruct((B, N, LD), _f32),
    )(h3, W_mlp1, b1)
    return out
